# baseline (speedup 1.0000x reference)
import jax
import jax.numpy as jnp
from jax.experimental import pallas as pl
from jax.experimental.pallas import tpu as pltpu

NUM_ATOMS = 1024
HID = 768
EMB = 768
ANT_LEN = 4
BATCH = 1024

BB = 256


def _body(rep_ref, mask_ref, wih_hbm, whh_hbm, bih_ref, bhh_ref,
          hw_hbm, hb_ref, emb_hbm, out_ref,
          wih_ref, whh_ref, hw_ref, emb_ref,
          sem_wih, sem_whh, sem_hw, sem_emb):
    i = pl.program_id(0)
    cp_wih = pltpu.make_async_copy(wih_hbm, wih_ref, sem_wih)
    cp_whh = pltpu.make_async_copy(whh_hbm, whh_ref, sem_whh)
    cp_hw = pltpu.make_async_copy(hw_hbm, hw_ref, sem_hw)
    cp_emb = pltpu.make_async_copy(emb_hbm, emb_ref, sem_emb)

    @pl.when(i == 0)
    def _():
        cp_wih.start()
        cp_hw.start()
        cp_whh.start()
        cp_emb.start()

    rep = rep_ref[...]
    mask = mask_ref[...]
    b_ih = bih_ref[...]
    b_hh = bhh_ref[...]
    hb = hb_ref[...]

    def mm_t(a, b):
        return jax.lax.dot_general(a, b, (((1,), (1,)), ((), ())),
                                   preferred_element_type=jnp.float32)

    n_iota = jax.lax.broadcasted_iota(jnp.int32, (1, NUM_ATOMS), 1)
    col0 = n_iota == 0
    neg_inf = jnp.float32(-jnp.inf)

    @pl.when(i == 0)
    def _():
        cp_wih.wait()
    wih = wih_ref[...]
    gi = mm_t(rep, wih) + b_ih
    gh = b_hh

    @pl.when(i == 0)
    def _():
        cp_hw.wait()
    hw = hw_ref[...]

    @pl.when(i == 0)
    def _():
        cp_whh.wait()
    whh = whh_ref[...]

    @pl.when(i == 0)
    def _():
        cp_emb.wait()
    emb = emb_ref[...]

    prev_ind = None
    h = None
    for j in range(ANT_LEN):
        r = jax.nn.sigmoid(gi[:, :EMB] + gh[:, :EMB])
        z = jax.nn.sigmoid(gi[:, EMB:2 * EMB] + gh[:, EMB:2 * EMB])
        n = jnp.tanh(gi[:, 2 * EMB:] + r * gh[:, 2 * EMB:])
        h = (1.0 - z) * n if j == 0 else (1.0 - z) * n + z * h

        logits = mm_t(h, hw) + hb

        if j + 1 < ANT_LEN:
            gh = mm_t(h, whh) + b_hh

        if j == 0:
            empty = jnp.sum(mask, axis=-1, keepdims=True) == 0.0
            mask = jnp.where(col0 & empty, 1.0, mask)
        else:
            mask = jnp.where(prev_ind == 0, 0.0, mask)
            mask = jnp.where(col0, 1.0, mask)

        masked = jnp.where(mask != 0.0, logits, neg_inf)
        mx = jnp.max(masked, axis=-1, keepdims=True)
        cand = jnp.where(masked == mx, n_iota, NUM_ATOMS)
        ind = jnp.min(cand, axis=-1, keepdims=True)
        sel = n_iota == ind
        onehot = sel.astype(jnp.float32)
        out_ref[:, j, :] = onehot
        mask = jnp.where(sel, 0.0, mask)
        prev_ind = ind

        if j + 1 < ANT_LEN:
            wsum = jnp.dot(onehot, emb, preferred_element_type=jnp.float32)
            gi = mm_t(rep + wsum, wih) + b_ih


@jax.jit
def _run(rep, x_, wih_t, whh_t, b_ih, b_hh, hw_t, hb, emb):
    grid = (BATCH // BB,)
    const = lambda i: (0, 0)
    hbm = pl.BlockSpec(memory_space=pl.ANY)
    return pl.pallas_call(
        _body,
        grid=grid,
        in_specs=[
            pl.BlockSpec((BB, HID), lambda i: (i, 0)),
            pl.BlockSpec((BB, NUM_ATOMS), lambda i: (i, 0)),
            hbm,
            hbm,
            pl.BlockSpec((1, 3 * EMB), const),
            pl.BlockSpec((1, 3 * EMB), const),
            hbm,
            pl.BlockSpec((1, NUM_ATOMS), const),
            hbm,
        ],
        out_specs=pl.BlockSpec((BB, ANT_LEN, NUM_ATOMS), lambda i: (i, 0, 0)),
        out_shape=jax.ShapeDtypeStruct((BATCH, ANT_LEN, NUM_ATOMS), jnp.float32),
        scratch_shapes=[
            pltpu.VMEM((3 * EMB, HID), jnp.float32),
            pltpu.VMEM((3 * EMB, EMB), jnp.float32),
            pltpu.VMEM((NUM_ATOMS, EMB), jnp.float32),
            pltpu.VMEM((NUM_ATOMS, EMB), jnp.float32),
            pltpu.SemaphoreType.DMA,
            pltpu.SemaphoreType.DMA,
            pltpu.SemaphoreType.DMA,
            pltpu.SemaphoreType.DMA,
        ],
        compiler_params=pltpu.CompilerParams(
            dimension_semantics=("arbitrary",)),
    )(rep, x_, wih_t, whh_t, b_ih, b_hh, hw_t, hb, emb)


def kernel(representation_emb, x_, W_ih, W_hh, b_ih, b_hh, head_w, head_b,
           atom_embedding):
    return _run(representation_emb, x_,
                W_ih, W_hh,
                b_ih.reshape(1, -1), b_hh.reshape(1, -1),
                head_w, head_b.reshape(1, -1),
                atom_embedding)


# speedup vs baseline: 1.0323x; 1.0323x over previous
import jax
import jax.numpy as jnp
from jax.experimental import pallas as pl
from jax.experimental.pallas import tpu as pltpu

NUM_ATOMS = 1024
HID = 768
EMB = 768
ANT_LEN = 4
BATCH = 1024

BB = 256


def _body(rep_ref, mask_ref, wih_hbm, whh_hbm, bih_ref, bhh_ref,
          hw_hbm, hb_ref, emb_hbm, out_ref,
          wih_ref, whh_ref, hw_ref, emb_ref,
          sem_wih, sem_whh, sem_hw, sem_emb):
    i = pl.program_id(0)
    cp_wih = pltpu.make_async_copy(wih_hbm, wih_ref, sem_wih)
    cp_whh = pltpu.make_async_copy(whh_hbm, whh_ref, sem_whh)
    cp_hw = pltpu.make_async_copy(hw_hbm, hw_ref, sem_hw)
    cp_emb = pltpu.make_async_copy(emb_hbm, emb_ref, sem_emb)

    @pl.when(i == 0)
    def _():
        cp_wih.start()
        cp_hw.start()
        cp_whh.start()
        cp_emb.start()

    rep = rep_ref[...]
    mask = mask_ref[...]
    b_ih = bih_ref[...]
    b_hh = bhh_ref[...]
    hb = hb_ref[...]

    def mm_t(a, b):
        return jax.lax.dot_general(a, b, (((1,), (1,)), ((), ())),
                                   preferred_element_type=jnp.float32)

    n_iota = jax.lax.broadcasted_iota(jnp.int32, (1, NUM_ATOMS), 1)
    col0 = n_iota == 0
    neg_inf = jnp.float32(-jnp.inf)

    @pl.when(i == 0)
    def _():
        cp_wih.wait()
    wih = wih_ref[...]
    gi = mm_t(rep, wih) + b_ih
    gh = b_hh

    @pl.when(i == 0)
    def _():
        cp_hw.wait()
    hw = hw_ref[...]

    prev_ind = None
    h = None
    for j in range(ANT_LEN):
        r = jax.nn.sigmoid(gi[:, :EMB] + gh[:, :EMB])
        z = jax.nn.sigmoid(gi[:, EMB:2 * EMB] + gh[:, EMB:2 * EMB])
        n = jnp.tanh(gi[:, 2 * EMB:] + r * gh[:, 2 * EMB:])
        h = (1.0 - z) * n if j == 0 else (1.0 - z) * n + z * h

        logits = mm_t(h, hw) + hb

        if j == 0:
            @pl.when(i == 0)
            def _():
                cp_whh.wait()
                cp_emb.wait()
            whh = whh_ref[...]
            emb = emb_ref[...]

        if j + 1 < ANT_LEN:
            gh = mm_t(h, whh) + b_hh

        if j == 0:
            empty = jnp.sum(mask, axis=-1, keepdims=True) == 0.0
            mask = jnp.where(col0 & empty, 1.0, mask)
        else:
            mask = jnp.where(prev_ind == 0, 0.0, mask)
            mask = jnp.where(col0, 1.0, mask)

        masked = jnp.where(mask != 0.0, logits, neg_inf)
        mx = jnp.max(masked, axis=-1, keepdims=True)
        cand = jnp.where(masked == mx, n_iota, NUM_ATOMS)
        ind = jnp.min(cand, axis=-1, keepdims=True)
        sel = n_iota == ind
        onehot = sel.astype(jnp.float32)
        out_ref[:, j, :] = onehot
        mask = jnp.where(sel, 0.0, mask)
        prev_ind = ind

        if j + 1 < ANT_LEN:
            wsum = jnp.dot(onehot, emb, preferred_element_type=jnp.float32)
            gi = mm_t(rep + wsum, wih) + b_ih


@jax.jit
def _run(rep, x_, wih_t, whh_t, b_ih, b_hh, hw_t, hb, emb):
    grid = (BATCH // BB,)
    const = lambda i: (0, 0)
    hbm = pl.BlockSpec(memory_space=pl.ANY)
    return pl.pallas_call(
        _body,
        grid=grid,
        in_specs=[
            pl.BlockSpec((BB, HID), lambda i: (i, 0)),
            pl.BlockSpec((BB, NUM_ATOMS), lambda i: (i, 0)),
            hbm,
            hbm,
            pl.BlockSpec((1, 3 * EMB), const),
            pl.BlockSpec((1, 3 * EMB), const),
            hbm,
            pl.BlockSpec((1, NUM_ATOMS), const),
            hbm,
        ],
        out_specs=pl.BlockSpec((BB, ANT_LEN, NUM_ATOMS), lambda i: (i, 0, 0)),
        out_shape=jax.ShapeDtypeStruct((BATCH, ANT_LEN, NUM_ATOMS), jnp.float32),
        scratch_shapes=[
            pltpu.VMEM((3 * EMB, HID), jnp.float32),
            pltpu.VMEM((3 * EMB, EMB), jnp.float32),
            pltpu.VMEM((NUM_ATOMS, EMB), jnp.float32),
            pltpu.VMEM((NUM_ATOMS, EMB), jnp.float32),
            pltpu.SemaphoreType.DMA,
            pltpu.SemaphoreType.DMA,
            pltpu.SemaphoreType.DMA,
            pltpu.SemaphoreType.DMA,
        ],
        compiler_params=pltpu.CompilerParams(
            dimension_semantics=("arbitrary",)),
    )(rep, x_, wih_t, whh_t, b_ih, b_hh, hw_t, hb, emb)


def kernel(representation_emb, x_, W_ih, W_hh, b_ih, b_hh, head_w, head_b,
           atom_embedding):
    return _run(representation_emb, x_,
                W_ih, W_hh,
                b_ih.reshape(1, -1), b_hh.reshape(1, -1),
                head_w, head_b.reshape(1, -1),
                atom_embedding)
